# Initial kernel scaffold; baseline (speedup 1.0000x reference)
import jax
import jax.numpy as jnp
from jax.experimental import pallas as pl


def kernel(x, embedding):
    raise NotImplementedError("write your pallas kernel here")


# speedup vs baseline: 2.9957x; 2.9957x over previous
import functools

import jax
import jax.numpy as jnp
from jax import lax
from jax.experimental import pallas as pl
from jax.experimental.pallas import tpu as pltpu
from jax.experimental.pallas import tpu_sc as plsc

B, S = 16384, 200
V, D = 50, 16
N = B * S
NW = 32
PER_W = N // NW
SUB = 128
NSUB = 16
CHUNK = SUB * NSUB
NCHUNKS = PER_W // CHUNK


def _lookup_body(x_hbm, emb_hbm, out_hbm, idx_v, rows_v, sem):
    cid = lax.axis_index("c")
    sid = lax.axis_index("s")
    wid = sid * 2 + cid
    base_row = wid * (PER_W // SUB)

    @pl.loop(0, NCHUNKS)
    def _chunk(ci):
        crow = base_row + ci * NSUB
        pltpu.sync_copy(x_hbm.at[pl.ds(crow, NSUB)], idx_v)
        copies = [
            pltpu.async_copy(
                emb_hbm.at[idx_v.at[j]],
                rows_v.at[pl.ds(j * SUB, SUB)],
                sem,
            )
            for j in range(NSUB)
        ]
        for c in copies:
            c.wait()
        pltpu.sync_copy(rows_v, out_hbm.at[pl.ds(crow * SUB, CHUNK)])


@jax.jit
def _lookup(x2, embedding):
    mesh = plsc.VectorSubcoreMesh(core_axis_name="c", subcore_axis_name="s")
    return pl.kernel(
        _lookup_body,
        out_type=jax.ShapeDtypeStruct((N, D), jnp.float32),
        mesh=mesh,
        scratch_types=[
            pltpu.VMEM((NSUB, SUB), jnp.int32),
            pltpu.VMEM((CHUNK, D), jnp.float32),
            pltpu.SemaphoreType.DMA,
        ],
        compiler_params=pltpu.CompilerParams(use_tc_tiling_on_sc=False),
    )(x2, embedding)


def kernel(x, embedding):
    x2 = x.reshape(N // SUB, SUB).astype(jnp.int32)
    out = _lookup(x2, embedding)
    return out.reshape(B, S, D)


# speedup vs baseline: 3.0089x; 1.0044x over previous
import jax
import jax.numpy as jnp
from jax import lax
from jax.experimental import pallas as pl
from jax.experimental.pallas import tpu as pltpu
from jax.experimental.pallas import tpu_sc as plsc

B, S = 16384, 200
V, D = 50, 16
N = B * S
NW = 32
PER_W = N // NW
SUB = 128
NSUB = 16
CHUNK = SUB * NSUB
NCHUNKS = PER_W // CHUNK


def _lookup_body(x_hbm, emb_hbm, out_hbm, idx_v, rows_v, sem_idx, sem_g, sem_st):
    cid = lax.axis_index("c")
    sid = lax.axis_index("s")
    wid = sid * 2 + cid
    base_row = wid * (PER_W // SUB)

    def start_idx(c, b):
        pltpu.async_copy(
            x_hbm.at[pl.ds(base_row + c * NSUB, NSUB)], idx_v.at[b], sem_idx.at[b]
        )

    start_idx(0, 0)
    start_idx(1, 1)

    @pl.loop(0, NCHUNKS, step=2)
    def _super(g):
        for b in range(2):
            c = g + b

            @pl.when(c >= 2)
            def _():
                pltpu.make_async_copy(
                    rows_v.at[b], out_hbm.at[pl.ds(0, CHUNK)], sem_st.at[b]
                ).wait()

            pltpu.make_async_copy(
                x_hbm.at[pl.ds(0, NSUB)], idx_v.at[b], sem_idx.at[b]
            ).wait()
            gathers = [
                pltpu.async_copy(
                    emb_hbm.at[idx_v.at[b].at[j]],
                    rows_v.at[b].at[pl.ds(j * SUB, SUB)],
                    sem_g.at[b],
                )
                for j in range(NSUB)
            ]

            for cpy in gathers:
                cpy.wait()

            @pl.when(c + 2 < NCHUNKS)
            def _():
                start_idx(c + 2, b)
            pltpu.async_copy(
                rows_v.at[b],
                out_hbm.at[pl.ds((base_row + c * NSUB) * SUB, CHUNK)],
                sem_st.at[b],
            )

    for b in range(2):
        pltpu.make_async_copy(
            rows_v.at[b], out_hbm.at[pl.ds(0, CHUNK)], sem_st.at[b]
        ).wait()


@jax.jit
def _lookup(x2, embedding):
    mesh = plsc.VectorSubcoreMesh(core_axis_name="c", subcore_axis_name="s")
    return pl.kernel(
        _lookup_body,
        out_type=jax.ShapeDtypeStruct((N, D), jnp.float32),
        mesh=mesh,
        scratch_types=[
            pltpu.VMEM((2, NSUB, SUB), jnp.int32),
            pltpu.VMEM((2, CHUNK, D), jnp.float32),
            pltpu.SemaphoreType.DMA((2,)),
            pltpu.SemaphoreType.DMA((2,)),
            pltpu.SemaphoreType.DMA((2,)),
        ],
        compiler_params=pltpu.CompilerParams(use_tc_tiling_on_sc=False),
    )(x2, embedding)


def kernel(x, embedding):
    x2 = x.reshape(N // SUB, SUB).astype(jnp.int32)
    out = _lookup(x2, embedding)
    return out.reshape(B, S, D)


# speedup vs baseline: 4.8661x; 1.6172x over previous
import jax
import jax.numpy as jnp
from jax import lax
from jax.experimental import pallas as pl
from jax.experimental.pallas import tpu as pltpu
from jax.experimental.pallas import tpu_sc as plsc

B, S = 16384, 200
V, D = 50, 16
N = B * S
NW = 32
PER_W = N // NW
CHUNK = 2048
NCHUNKS = PER_W // CHUNK
GROUPS = CHUNK // 16
L = 16


def _lookup_body(x_hbm, emb_hbm, out_hbm, emb_v, idx_v, rows_v, sem_idx, sem_st):
    cid = lax.axis_index("c")
    sid = lax.axis_index("s")
    wid = sid * 2 + cid
    base = wid * PER_W

    pltpu.sync_copy(emb_hbm, emb_v)

    def start_idx(c, b):
        pltpu.async_copy(
            x_hbm.at[pl.ds(base + c * CHUNK, CHUNK)], idx_v.at[b], sem_idx.at[b]
        )

    start_idx(0, 0)
    start_idx(1, 1)

    lane16 = lax.iota(jnp.int32, L) * D

    @pl.loop(0, NCHUNKS, step=2)
    def _super(g0):
        for b in range(2):
            c = g0 + b

            @pl.when(c >= 2)
            def _():
                pltpu.make_async_copy(
                    rows_v.at[b], out_hbm.at[pl.ds(0, CHUNK * D)], sem_st.at[b]
                ).wait()

            pltpu.make_async_copy(
                x_hbm.at[pl.ds(0, CHUNK)], idx_v.at[b], sem_idx.at[b]
            ).wait()

            idx_ref = idx_v.at[b]
            rows_ref = rows_v.at[b]

            @pl.loop(0, GROUPS, unroll=4)
            def _grp(g):
                idxv = idx_ref[pl.ds(g * L, L)]
                gbase = idxv * D
                sbase = lane16 + g * (L * D)
                for d in range(D):
                    vals = plsc.load_gather(emb_v, [gbase + d])
                    plsc.store_scatter(rows_ref, [sbase + d], vals)

            @pl.when(c + 2 < NCHUNKS)
            def _():
                start_idx(c + 2, b)

            pltpu.async_copy(
                rows_v.at[b],
                out_hbm.at[pl.ds((base + c * CHUNK) * D, CHUNK * D)],
                sem_st.at[b],
            )

    for b in range(2):
        pltpu.make_async_copy(
            rows_v.at[b], out_hbm.at[pl.ds(0, CHUNK * D)], sem_st.at[b]
        ).wait()


@jax.jit
def _lookup(x_flat, emb_flat):
    mesh = plsc.VectorSubcoreMesh(core_axis_name="c", subcore_axis_name="s")
    return pl.kernel(
        _lookup_body,
        out_type=jax.ShapeDtypeStruct((N * D,), jnp.float32),
        mesh=mesh,
        scratch_types=[
            pltpu.VMEM((V * D,), jnp.float32),
            pltpu.VMEM((2, CHUNK), jnp.int32),
            pltpu.VMEM((2, CHUNK * D), jnp.float32),
            pltpu.SemaphoreType.DMA((2,)),
            pltpu.SemaphoreType.DMA((2,)),
        ],
        compiler_params=pltpu.CompilerParams(
            use_tc_tiling_on_sc=False, needs_layout_passes=False
        ),
    )(x_flat, emb_flat)


def kernel(x, embedding):
    x_flat = x.reshape(N).astype(jnp.int32)
    out = _lookup(x_flat, embedding.reshape(V * D))
    return out.reshape(B, S, D)


# speedup vs baseline: 13.4939x; 2.7730x over previous
import jax
import jax.numpy as jnp
from jax import lax
from jax.experimental import pallas as pl
from jax.experimental.pallas import tpu as pltpu
from jax.experimental.pallas import tpu_sc as plsc

B, S = 16384, 200
V, D = 50, 16
N = B * S
NW = 32
L = 16
NTILES = S * (D // 8) * (B // 128)
PER_W = NTILES // NW
TPC = 32
NCHUNKS = PER_W // TPC
CIDX = TPC * 128
GROUPS = CIDX // L
COUT = TPC * 1024


def _lookup_body(xt_hbm, emb_hbm, out_hbm, emb_v, idx_v, rows_v, sem_idx, sem_st):
    cid = lax.axis_index("c")
    sid = lax.axis_index("s")
    wid = sid * 2 + cid
    t_base = wid * PER_W

    pltpu.sync_copy(emb_hbm, emb_v)

    def start_idx(c, b):
        t0 = t_base + c * TPC
        s = t0 >> 8
        nt0 = t0 & 127
        pltpu.async_copy(
            xt_hbm.at[pl.ds(s * B + nt0 * 128, CIDX)], idx_v.at[b], sem_idx.at[b]
        )

    start_idx(0, 0)
    start_idx(1, 1)

    lane = lax.iota(jnp.int32, L)

    @pl.loop(0, NCHUNKS, step=2)
    def _super(g0):
        for b in range(2):
            c = g0 + b
            t0 = t_base + c * TPC
            dt = (t0 >> 7) & 1

            @pl.when(c >= 2)
            def _():
                pltpu.make_async_copy(
                    rows_v.at[b], out_hbm.at[pl.ds(0, COUT)], sem_st.at[b]
                ).wait()

            pltpu.make_async_copy(
                xt_hbm.at[pl.ds(0, CIDX)], idx_v.at[b], sem_idx.at[b]
            ).wait()

            idx_ref = idx_v.at[b]
            rows_ref = rows_v.at[b]

            @pl.loop(0, GROUPS, unroll=4)
            def _grp(g):
                idxv = idx_ref[pl.ds(g * L, L)]
                gbase = idxv * D + dt * 8
                sbase = lane + ((g >> 3) << 10) + ((g & 7) << 4)
                for di in range(8):
                    vals = plsc.load_gather(emb_v, [gbase + di])
                    plsc.store_scatter(rows_ref, [sbase + (di << 7)], vals)

            @pl.when(c + 2 < NCHUNKS)
            def _():
                start_idx(c + 2, b)

            pltpu.async_copy(
                rows_v.at[b],
                out_hbm.at[pl.ds(t0 * 1024, COUT)],
                sem_st.at[b],
            )

    for b in range(2):
        pltpu.make_async_copy(
            rows_v.at[b], out_hbm.at[pl.ds(0, COUT)], sem_st.at[b]
        ).wait()


@jax.jit
def _lookup(xt_flat, emb_flat):
    mesh = plsc.VectorSubcoreMesh(core_axis_name="c", subcore_axis_name="s")
    return pl.kernel(
        _lookup_body,
        out_type=jax.ShapeDtypeStruct((N * D,), jnp.float32),
        mesh=mesh,
        scratch_types=[
            pltpu.VMEM((V * D,), jnp.float32),
            pltpu.VMEM((2, CIDX), jnp.int32),
            pltpu.VMEM((2, COUT), jnp.float32),
            pltpu.SemaphoreType.DMA((2,)),
            pltpu.SemaphoreType.DMA((2,)),
        ],
        compiler_params=pltpu.CompilerParams(
            use_tc_tiling_on_sc=False, needs_layout_passes=False
        ),
    )(xt_flat, emb_flat)


def kernel(x, embedding):
    xt_flat = jnp.transpose(x).reshape(N).astype(jnp.int32)
    out = _lookup(xt_flat, embedding.reshape(V * D))
    out5 = out.reshape(S, D // 8, B // 128, 8, 128)
    return jnp.transpose(out5, (2, 4, 0, 1, 3)).reshape(B, S, D)


# speedup vs baseline: 33.6875x; 2.4965x over previous
import jax
import jax.numpy as jnp
from jax import lax
from jax.experimental import pallas as pl
from jax.experimental.pallas import tpu as pltpu
from jax.experimental.pallas import tpu_sc as plsc

B, S = 16384, 200
V, D = 50, 16
N = B * S
NW = 32
L = 16
NTILES = S * (D // 8) * (B // 128)
PER_W = NTILES // NW
TPC = 32
NCHUNKS = PER_W // TPC
CIDX = TPC * 128
GROUPS = CIDX // L
COUT = TPC * 1024


def _lookup_body(xt_hbm, emb_hbm, out_hbm, emb_v, idx_v, rows_v, sem_idx, sem_st):
    cid = lax.axis_index("c")
    sid = lax.axis_index("s")
    wid = sid * 2 + cid
    t_base = wid * PER_W

    pltpu.sync_copy(emb_hbm, emb_v)

    def start_idx(c, b):
        t0 = t_base + c * TPC
        s = t0 >> 8
        nt0 = t0 & 127
        pltpu.async_copy(
            xt_hbm.at[pl.ds(s * B + nt0 * 128, CIDX)], idx_v.at[b], sem_idx.at[b]
        )

    start_idx(0, 0)
    start_idx(1, 1)

    lane = lax.iota(jnp.int32, L)

    @pl.loop(0, NCHUNKS, step=2)
    def _super(g0):
        for b in range(2):
            c = g0 + b
            t0 = t_base + c * TPC
            dt = (t0 >> 7) & 1

            @pl.when(c >= 2)
            def _():
                pltpu.make_async_copy(
                    rows_v.at[b], out_hbm.at[pl.ds(0, COUT)], sem_st.at[b]
                ).wait()

            pltpu.make_async_copy(
                xt_hbm.at[pl.ds(0, CIDX)], idx_v.at[b], sem_idx.at[b]
            ).wait()

            idx_ref = idx_v.at[b]
            rows_ref = rows_v.at[b]

            @plsc.parallel_loop(0, GROUPS, unroll=4)
            def _grp(g):
                idxv = idx_ref[pl.ds(g * L, L)]
                gbase = idxv * D + dt * 8
                sbase = lane + ((g >> 3) << 10) + ((g & 7) << 4)
                for di in range(8):
                    vals = plsc.load_gather(emb_v, [gbase + di])
                    plsc.store_scatter(rows_ref, [sbase + (di << 7)], vals)

            @pl.when(c + 2 < NCHUNKS)
            def _():
                start_idx(c + 2, b)

            pltpu.async_copy(
                rows_v.at[b],
                out_hbm.at[pl.ds(t0 * 1024, COUT)],
                sem_st.at[b],
            )

    for b in range(2):
        pltpu.make_async_copy(
            rows_v.at[b], out_hbm.at[pl.ds(0, COUT)], sem_st.at[b]
        ).wait()


@jax.jit
def _lookup(xt_flat, emb_flat):
    mesh = plsc.VectorSubcoreMesh(core_axis_name="c", subcore_axis_name="s")
    return pl.kernel(
        _lookup_body,
        out_type=jax.ShapeDtypeStruct((N * D,), jnp.float32),
        mesh=mesh,
        scratch_types=[
            pltpu.VMEM((V * D,), jnp.float32),
            pltpu.VMEM((2, CIDX), jnp.int32),
            pltpu.VMEM((2, COUT), jnp.float32),
            pltpu.SemaphoreType.DMA((2,)),
            pltpu.SemaphoreType.DMA((2,)),
        ],
        compiler_params=pltpu.CompilerParams(
            use_tc_tiling_on_sc=False, needs_layout_passes=False
        ),
    )(xt_flat, emb_flat)


def kernel(x, embedding):
    xt_flat = jnp.transpose(x).reshape(N).astype(jnp.int32)
    out = _lookup(xt_flat, embedding.reshape(V * D))
    out5 = out.reshape(S, D // 8, B // 128, 8, 128)
    return jnp.transpose(out5, (2, 4, 0, 1, 3)).reshape(B, S, D)
